# baseline (speedup 1.0000x reference)
import functools

import jax
import jax.numpy as jnp
from jax import lax
from jax.experimental import pallas as pl
from jax.experimental.pallas import tpu as pltpu
from jax.experimental.pallas import tpu_sc as plsc

D_MODEL = 64
D_PAD = 128
SCALE = 8.0

NC = 2
NS = 16
NW = NC * NS

SEQ_PAD = 56
X_PAD = 128
NBUF = 4


def _build(rows_w, n_chunks):
    mesh = plsc.VectorSubcoreMesh(core_axis_name="c", subcore_axis_name="s")

    @functools.partial(
        pl.kernel,
        out_type=jax.ShapeDtypeStruct(
            (rows_w * NW, SEQ_PAD, D_MODEL), jnp.float32
        ),
        mesh=mesh,
        compiler_params=pltpu.CompilerParams(use_tc_tiling_on_sc=True),
        scratch_types=[
            pltpu.VMEM((rows_w, X_PAD), jnp.int32),
            pltpu.VMEM((NBUF, SEQ_PAD, D_PAD), jnp.float32),
            pltpu.VMEM((NBUF, SEQ_PAD, D_MODEL), jnp.float32),
            pltpu.SemaphoreType.DMA((NBUF,)),
            pltpu.SemaphoreType.DMA((NBUF,)),
        ],
    )
    def emb(x_hbm, tab_hbm, out_hbm, idx_v, rows_v, outb_v, gsem, osem):
        wid = lax.axis_index("s") * NC + lax.axis_index("c")
        rbase = wid * rows_w
        pltpu.sync_copy(x_hbm.at[pl.ds(rbase, rows_w)], idx_v)

        def fire_gather(g, b):
            pltpu.async_copy(
                tab_hbm.at[idx_v.at[g, pl.ds(0, SEQ_PAD)]],
                rows_v.at[b],
                gsem.at[b],
            )

        def drain_gather(b):
            pltpu.make_async_copy(
                tab_hbm.at[idx_v.at[0, pl.ds(0, SEQ_PAD)]],
                rows_v.at[b],
                gsem.at[b],
            ).wait()

        def fire_out(g, b):
            pltpu.async_copy(
                outb_v.at[b],
                out_hbm.at[rbase + g],
                osem.at[b],
            )

        def wait_out(b):
            pltpu.make_async_copy(
                outb_v.at[b], out_hbm.at[0], osem.at[b]
            ).wait()

        for g in range(NBUF - 1):
            fire_gather(g, g)

        def chunk_iter(t, carry):
            for b in range(NBUF):
                g = t * NBUF + b
                drain_gather(b)

                @pl.when(g >= NBUF)
                def _w():
                    wait_out(b)

                def scale_row(r, c):
                    for k in range(D_MODEL // 16):
                        sl = pl.ds(16 * k, 16)
                        outb_v[b, r, sl] = rows_v[b, r, sl] * SCALE
                    return c

                lax.fori_loop(0, SEQ_PAD, scale_row, 0)
                fire_out(g, b)
                nb = (b + NBUF - 1) % NBUF

                @pl.when(g + NBUF - 1 < n_chunks)
                def _prep():
                    fire_gather(g + NBUF - 1, nb)

            return carry

        lax.fori_loop(0, n_chunks // NBUF, chunk_iter, 0)
        for b in range(NBUF):
            wait_out(b)

    return emb


def kernel(x, table):
    nrows, seq = x.shape
    assert nrows % NW == 0 and seq <= SEQ_PAD
    rows_w = nrows // NW
    n_chunks = rows_w
    assert n_chunks % NBUF == 0
    x_pad = jnp.pad(x.astype(jnp.int32), ((0, 0), (0, X_PAD - seq)))
    tab_pad = jnp.pad(table, ((0, 0), (0, D_PAD - D_MODEL)))
    out = _build(rows_w, n_chunks)(x_pad, tab_pad)
    return out[:, :seq, :]


# speedup vs baseline: 4.1669x; 4.1669x over previous
import functools

import jax
import jax.numpy as jnp
from jax import lax
from jax.experimental import pallas as pl
from jax.experimental.pallas import tpu as pltpu
from jax.experimental.pallas import tpu_sc as plsc

D_MODEL = 64
SCALE = 8.0

NC = 2
NS = 16
NW = NC * NS

CR = 4
NBUF = 4


def _build(seq, rows_w, n_chunks):
    mesh = plsc.VectorSubcoreMesh(core_axis_name="c", subcore_axis_name="s")

    @functools.partial(
        pl.kernel,
        out_type=jax.ShapeDtypeStruct((rows_w * NW, seq, D_MODEL), jnp.float32),
        mesh=mesh,
        compiler_params=pltpu.CompilerParams(use_tc_tiling_on_sc=False),
        scratch_types=[
            pltpu.VMEM((rows_w, seq), jnp.int32),
            pltpu.VMEM((NBUF, CR, seq, D_MODEL), jnp.float32),
            pltpu.SemaphoreType.DMA((NBUF,)),
            pltpu.SemaphoreType.DMA((NBUF,)),
        ],
    )
    def emb(x_hbm, tab_hbm, out_hbm, idx_v, rows_v, gsem, osem):
        wid = lax.axis_index("s") * NC + lax.axis_index("c")
        rbase = wid * rows_w
        pltpu.sync_copy(x_hbm.at[pl.ds(rbase, rows_w)], idx_v)

        def fire_gather(g, b):
            for j in range(CR):
                pltpu.async_copy(
                    tab_hbm.at[idx_v.at[g * CR + j]],
                    rows_v.at[b, j],
                    gsem.at[b],
                )

        def drain_gather(b):
            for j in range(CR):
                pltpu.make_async_copy(
                    tab_hbm.at[idx_v.at[j]], rows_v.at[b, j], gsem.at[b]
                ).wait()

        def fire_out(g, b):
            pltpu.async_copy(
                rows_v.at[b],
                out_hbm.at[pl.ds(rbase + g * CR, CR)],
                osem.at[b],
            )

        def wait_out(b):
            pltpu.make_async_copy(
                rows_v.at[b], out_hbm.at[pl.ds(0, CR)], osem.at[b]
            ).wait()

        for g in range(NBUF - 1):
            fire_gather(g, g)

        def chunk_iter(t, carry):
            for b in range(NBUF):
                g = t * NBUF + b
                drain_gather(b)

                def scale_row(r, c):
                    for j in range(CR):
                        for k in range(D_MODEL // 16):
                            sl = pl.ds(16 * k, 16)
                            rows_v[b, j, r, sl] = rows_v[b, j, r, sl] * SCALE
                    return c

                lax.fori_loop(0, seq, scale_row, 0)
                fire_out(g, b)
                nb = (b + NBUF - 1) % NBUF

                @pl.when(g + NBUF - 1 < n_chunks)
                def _prep():
                    @pl.when(g >= 1)
                    def _w():
                        wait_out(nb)

                    fire_gather(g + NBUF - 1, nb)

            return carry

        lax.fori_loop(0, n_chunks // NBUF, chunk_iter, 0)
        for b in range(NBUF):
            wait_out(b)

    return emb


NSPLIT = 4


def kernel(x, table):
    nrows, seq = x.shape
    assert nrows % (NW * NSPLIT) == 0
    part = nrows // NSPLIT
    rows_w = part // NW
    assert rows_w % CR == 0
    n_chunks = rows_w // CR
    assert n_chunks % NBUF == 0
    call = _build(seq, rows_w, n_chunks)
    xi = x.astype(jnp.int32)
    outs = [call(xi[i * part:(i + 1) * part], table) for i in range(NSPLIT)]
    return jnp.concatenate(outs, axis=0)
